# Initial kernel scaffold; baseline (speedup 1.0000x reference)
import jax
import jax.numpy as jnp
from jax.experimental import pallas as pl


def kernel(sim):
    raise NotImplementedError("write your pallas kernel here")


# speedup vs baseline: 9.8446x; 9.8446x over previous
import jax
import jax.numpy as jnp
from jax.experimental import pallas as pl

KK = 256
NROWS = 128
NCOLS = 32768
BLK = 8


def _tc_body(x_ref, o_ref):
    x = x_ref[...]
    b = jax.lax.bitcast_convert_type(x, jnp.int32)
    k = jnp.where(b < 0, b ^ jnp.int32(0x7FFFFFFF), b)

    def step(i, cur):
        stepv = jnp.int32(1) << (jnp.int32(30) - i)
        trial = cur + stepv
        cnt = jnp.sum((k >= trial).astype(jnp.int32), axis=1, keepdims=True)
        return jnp.where(cnt >= KK, trial, cur)

    cnt_pos = jnp.sum((k >= 0).astype(jnp.int32), axis=1, keepdims=True)
    cur0 = jnp.where(cnt_pos >= KK, jnp.int32(0), jnp.int32(-2147483648))
    T = jax.lax.fori_loop(0, 31, step, cur0, unroll=True)

    gt = k > T
    eq = k == T
    cnt_gt = jnp.sum(gt.astype(jnp.int32), axis=1, keepdims=True)
    m = KK - cnt_gt

    idx = jax.lax.broadcasted_iota(jnp.int32, (BLK, NCOLS), 1)

    def istep(i, cur):
        stepv = jnp.int32(1) << (jnp.int32(14) - i)
        trial = cur + stepv
        cnt = jnp.sum((eq & (idx <= trial)).astype(jnp.int32), axis=1,
                      keepdims=True)
        return jnp.where(cnt <= m - 1, trial, cur)

    jcur0 = jnp.full((BLK, 1), jnp.int32(-1))
    jmax = jax.lax.fori_loop(0, 15, istep, jcur0, unroll=True)

    win = gt | (eq & (idx <= jmax + 1))
    o_ref[...] = jnp.where(win, 0.0, jnp.inf).astype(jnp.float32)


def kernel(sim):
    return pl.pallas_call(
        _tc_body,
        grid=(NROWS // BLK,),
        in_specs=[pl.BlockSpec((BLK, NCOLS), lambda i: (i, 0))],
        out_specs=pl.BlockSpec((BLK, NCOLS), lambda i: (i, 0)),
        out_shape=jax.ShapeDtypeStruct((NROWS, NCOLS), jnp.float32),
    )(sim)
